# Initial kernel scaffold; baseline (speedup 1.0000x reference)
import jax
import jax.numpy as jnp
from jax.experimental import pallas as pl


def kernel(x, states, log_amplitudes):
    raise NotImplementedError("write your pallas kernel here")


# speedup vs baseline: 2.2078x; 2.2078x over previous
import functools

import jax
import jax.numpy as jnp
from jax import lax
from jax.experimental import pallas as pl
from jax.experimental.pallas import tpu as pltpu
from jax.experimental.pallas import tpu_sc as plsc

N = 1_000_000
B = 16_384
BLK = 16
NB = N // BLK
NC, NS = 2, 16
NW = NC * NS
BPW = B // NW
NG = BPW // 16

STEPS1 = [1 << k for k in range(15, -1, -1)]
STEPS2 = [16, 8, 4, 2, 1]


def _body(xq_hbm, bm_hbm, srows_hbm, arows_hbm, out_hbm,
          bm_v, xq_v, b_v, bc_v, srow_v, arow_v, out_v, sem_s, sem_a):
    wid = lax.axis_index("s") * jnp.int32(NC) + lax.axis_index("c")
    base = wid * jnp.int32(BPW)

    pltpu.sync_copy(bm_hbm, bm_v)
    pltpu.sync_copy(xq_hbm.at[pl.ds(base, BPW)], xq_v)

    def pass1(_, g):
        g16 = g * jnp.int32(16)
        xv = xq_v[pl.ds(g16, 16)]
        pos = jnp.zeros((16,), jnp.int32)
        for step in STEPS1:
            t = pos + step
            probe = jnp.minimum(t, NB) - 1
            v = plsc.load_gather(bm_v, [probe])
            pos = jnp.where((t <= NB) & (v < xv), t, pos)
        b_v[pl.ds(g16, 16)] = pos
        bc_v[pl.ds(g16, 16)] = jnp.minimum(pos, NB - 1)
        return g + jnp.int32(1)

    lax.fori_loop(0, NG, pass1, jnp.int32(0))

    cp_s = pltpu.async_copy(srows_hbm.at[bc_v], srow_v, sem_s)
    cp_a = pltpu.async_copy(arows_hbm.at[bc_v], arow_v, sem_a)
    cp_s.wait()
    cp_a.wait()

    def pass2(_, g):
        g16 = g * jnp.int32(16)
        xv = xq_v[pl.ds(g16, 16)]
        b = b_v[pl.ds(g16, 16)]
        bc = bc_v[pl.ds(g16, 16)]
        qidx = g16 + lax.iota(jnp.int32, 16)
        pos = jnp.zeros((16,), jnp.int32)
        for step in STEPS2:
            t = pos + step
            col = 2 * jnp.minimum(t, BLK) - 2
            v = plsc.load_gather(srow_v, [qidx, col])
            pos = jnp.where((t <= BLK) & (v < xv), t, pos)
        idxf = jnp.minimum(b * BLK + pos, N - 1)
        w = idxf - bc * BLK
        out_v[pl.ds(g16, 16)] = plsc.load_gather(arow_v, [qidx, w])
        return g + jnp.int32(1)

    lax.fori_loop(0, NG, pass2, jnp.int32(0))
    pltpu.sync_copy(out_v, out_hbm.at[pl.ds(base, BPW)])


@jax.jit
def _lookup(xq, bm, srows, arows):
    mesh = plsc.VectorSubcoreMesh(core_axis_name="c", subcore_axis_name="s",
                                  num_cores=NC, num_subcores=NS)
    return pl.kernel(
        _body,
        out_type=jax.ShapeDtypeStruct((B,), jnp.float32),
        mesh=mesh,
        scratch_types=[
            pltpu.VMEM((NB,), jnp.int32),
            pltpu.VMEM((BPW,), jnp.int32),
            pltpu.VMEM((BPW,), jnp.int32),
            pltpu.VMEM((BPW,), jnp.int32),
            pltpu.VMEM((BPW, 2 * BLK), jnp.int32),
            pltpu.VMEM((BPW, BLK), jnp.float32),
            pltpu.VMEM((BPW,), jnp.float32),
            pltpu.SemaphoreType.DMA,
            pltpu.SemaphoreType.DMA,
        ],
        compiler_params=pltpu.CompilerParams(needs_layout_passes=False,
                                             use_tc_tiling_on_sc=False),
    )(xq, bm, srows, arows)


def kernel(x, states, log_amplitudes):
    if x.ndim > 1:
        x = x[:, 0]
    xq = x.astype(jnp.int32)
    s_pairs = lax.bitcast_convert_type(states, jnp.int32)
    bm = s_pairs[BLK - 1::BLK, 0]
    srows = s_pairs.reshape(NB, 2 * BLK)
    arows = log_amplitudes.reshape(NB, BLK)
    return _lookup(xq, bm, srows, arows).reshape(B, 1)


# speedup vs baseline: 7.7134x; 3.4938x over previous
import functools

import jax
import jax.numpy as jnp
from jax import lax
from jax.experimental import pallas as pl
from jax.experimental.pallas import tpu as pltpu
from jax.experimental.pallas import tpu_sc as plsc

N = 1_000_000
B = 16_384
BLK = 16
NB = N // BLK
NC, NS = 2, 16
NW = NC * NS
BPW = B // NW
NG = BPW // 16

STEPS1 = [1 << k for k in range(15, -1, -1)]
STEPS2 = [16, 8, 4, 2, 1]


def _body(xq_hbm, bm_hbm, srows_hbm, arows_hbm, out_hbm,
          bm_v, xq_v, b_v, bc_v, srow_v, arow_v, out_v, sem_s, sem_a):
    wid = lax.axis_index("s") * jnp.int32(NC) + lax.axis_index("c")
    base = wid * jnp.int32(BPW)

    pltpu.sync_copy(bm_hbm, bm_v)
    pltpu.sync_copy(xq_hbm.at[pl.ds(base, BPW)], xq_v)

    def pass1(_, g):
        g16 = g * jnp.int32(16)
        xv = xq_v[pl.ds(g16, 16)]
        pos = jnp.zeros((16,), jnp.int32)
        for step in STEPS1:
            t = pos + step
            probe = jnp.minimum(t, NB) - 1
            v = plsc.load_gather(bm_v, [probe])
            pos = jnp.where((t <= NB) & (v < xv), t, pos)
        b_v[pl.ds(g16, 16)] = pos
        bc_v[pl.ds(g16, 16)] = jnp.minimum(pos, NB - 1)
        return g + jnp.int32(1)

    lax.fori_loop(0, NG, pass1, jnp.int32(0))

    cp_s = pltpu.async_copy(srows_hbm.at[bc_v], srow_v, sem_s)
    cp_a = pltpu.async_copy(arows_hbm.at[bc_v], arow_v, sem_a)
    cp_s.wait()
    cp_a.wait()

    def pass2(_, g):
        g16 = g * jnp.int32(16)
        xv = xq_v[pl.ds(g16, 16)]
        b = b_v[pl.ds(g16, 16)]
        bc = bc_v[pl.ds(g16, 16)]
        qidx = g16 + lax.iota(jnp.int32, 16)
        pos = jnp.zeros((16,), jnp.int32)
        for step in STEPS2:
            t = pos + step
            col = jnp.minimum(t, BLK) - 1
            v = plsc.load_gather(srow_v, [qidx, col])
            pos = jnp.where((t <= BLK) & (v < xv), t, pos)
        idxf = jnp.minimum(b * BLK + pos, N - 1)
        w = idxf - bc * BLK
        out_v[pl.ds(g16, 16)] = plsc.load_gather(arow_v, [qidx, w])
        return g + jnp.int32(1)

    lax.fori_loop(0, NG, pass2, jnp.int32(0))
    pltpu.sync_copy(out_v, out_hbm.at[pl.ds(base, BPW)])


@jax.jit
def _lookup(xq, bm, srows, arows):
    mesh = plsc.VectorSubcoreMesh(core_axis_name="c", subcore_axis_name="s",
                                  num_cores=NC, num_subcores=NS)
    return pl.kernel(
        _body,
        out_type=jax.ShapeDtypeStruct((B,), jnp.float32),
        mesh=mesh,
        scratch_types=[
            pltpu.VMEM((NB,), jnp.int32),
            pltpu.VMEM((BPW,), jnp.int32),
            pltpu.VMEM((BPW,), jnp.int32),
            pltpu.VMEM((BPW,), jnp.int32),
            pltpu.VMEM((BPW, BLK), jnp.int32),
            pltpu.VMEM((BPW, BLK), jnp.float32),
            pltpu.VMEM((BPW,), jnp.float32),
            pltpu.SemaphoreType.DMA,
            pltpu.SemaphoreType.DMA,
        ],
        compiler_params=pltpu.CompilerParams(needs_layout_passes=False,
                                             use_tc_tiling_on_sc=False),
    )(xq, bm, srows, arows)


def kernel(x, states, log_amplitudes):
    if x.ndim > 1:
        x = x[:, 0]
    xq = x.astype(jnp.int32)
    s32 = states.astype(jnp.int32)
    bm = s32[BLK - 1::BLK]
    srows = s32.reshape(NB, BLK)
    arows = log_amplitudes.reshape(NB, BLK)
    return _lookup(xq, bm, srows, arows).reshape(B, 1)
